# baseline (speedup 1.0000x reference)
import functools

import jax
import jax.numpy as jnp
from jax import lax
from jax.experimental import pallas as pl
from jax.experimental.pallas import tpu as pltpu
from jax.experimental.pallas import tpu_sc as plsc

_LAT_MIN, _LAT_MAX = -90.0, 90.0
_LON_MIN, _LON_MAX = -180.0, 180.0
_LAT_MEAN, _LAT_STD = 0.0, 1.0
_LON_MEAN, _LON_STD = 0.0, 1.0
_BINS = 100
_D = 128

_NC, _NS, _L = 2, 16, 16
_NW = _NC * _NS

_CROWS = 128
_NBUF = 6


@functools.partial(jax.jit, static_argnums=(1,))
def _sc_run(packed, batch):
    pairs_per_w = batch // _NW
    rows_per_w = 2 * pairs_per_w
    chunks = rows_per_w // _CROWS
    vecs_per_chunk = _CROWS // _L
    crs_rows_per_w = 2 * pairs_per_w // _D
    crs_rows = 2 * batch // _D
    seg = _BINS + 4
    lat_row0 = crs_rows
    lon_row0 = crs_rows + seg
    edge_row0 = crs_rows + 2 * seg

    def body(packed_hbm, out_hbm, crs_v, edges_v, idx_v, buf_v, tbl_sp,
             ssems, gsems, wsems):
        cid = lax.axis_index("c")
        sid = lax.axis_index("s")
        wid = sid * _NC + cid
        rbase = wid * rows_per_w

        @pl.when(sid == 0)
        def _stage_lat():
            pltpu.async_copy(packed_hbm.at[pl.ds(lat_row0, seg)],
                             tbl_sp.at[pl.ds(0, seg)], ssems[0])

        @pl.when(sid == 1)
        def _stage_lon():
            pltpu.async_copy(packed_hbm.at[pl.ds(lon_row0, seg)],
                             tbl_sp.at[pl.ds(seg, seg)], ssems[0])

        ccrs = pltpu.async_copy(
            packed_hbm.at[pl.ds(wid * crs_rows_per_w, crs_rows_per_w)],
            crs_v, ssems[1])
        cedg = pltpu.async_copy(packed_hbm.at[pl.ds(edge_row0, 8)],
                                edges_v, ssems[2])
        ccrs.wait()
        cedg.wait()

        lane = lax.iota(jnp.int32, _L)
        parity = lane & 1
        perm = lane ^ 1
        meanv = jnp.where(parity == 0, _LAT_MEAN, _LON_MEAN).astype(jnp.float32)
        inv_stdv = jnp.where(parity == 0, 1.0 / _LAT_STD,
                             1.0 / _LON_STD).astype(jnp.float32)
        minv = jnp.where(parity == 0, _LAT_MIN, _LON_MIN).astype(jnp.float32)
        inv_stepv = jnp.where(
            parity == 0,
            (_BINS - 2) / (_LAT_MAX - _LAT_MIN),
            (_BINS - 2) / (_LON_MAX - _LON_MIN),
        ).astype(jnp.float32)
        toff = parity * seg

        def compute(j, carry):
            flat = j * _L + perm
            x = plsc.load_gather(crs_v, [flat >> 7, flat & 127])
            x = (x - meanv) * inv_stdv
            q = (x - minv) * inv_stepv
            g = jnp.clip(q.astype(jnp.int32) + 1, 0, _BINS - 1)
            lo = plsc.load_gather(edges_v, [parity, jnp.maximum(g - 1, 0)])
            hi = plsc.load_gather(edges_v, [parity, jnp.minimum(g, _BINS - 2)])
            dec = ((g >= 1) & (x < lo)).astype(jnp.int32)
            inc = ((g <= _BINS - 2) & (x >= hi)).astype(jnp.int32)
            t = g - dec + inc + toff
            idx_v[j // vecs_per_chunk, pl.ds((j % vecs_per_chunk) * _L, _L)] = t
            return carry

        lax.fori_loop(0, chunks * vecs_per_chunk, compute, 0)

        @pl.when(sid < 2)
        def _wait_table():
            pltpu.make_async_copy(packed_hbm.at[pl.ds(lat_row0, seg)],
                                  tbl_sp.at[pl.ds(0, seg)], ssems[0]).wait()

        plsc.subcore_barrier()

        def start_gather(k):
            return pltpu.async_copy(tbl_sp.at[idx_v.at[k]],
                                    buf_v.at[k % _NBUF], gsems[k % _NBUF])

        def start_write(k):
            return pltpu.async_copy(buf_v.at[k % _NBUF],
                                    out_hbm.at[pl.ds(rbase + k * _CROWS, _CROWS)],
                                    wsems[k % _NBUF])

        gc = {k: start_gather(k) for k in range(min(_NBUF, chunks))}
        wc = {}
        for k in range(chunks):
            gc[k].wait()
            wc[k] = start_write(k)
            nk = k + _NBUF
            if nk < chunks:
                wc[k].wait()
                gc[nk] = start_gather(nk)
        for k in range(max(0, chunks - _NBUF), chunks):
            wc[k].wait()

    grid_kernel = pl.kernel(
        body,
        out_type=jax.ShapeDtypeStruct((2 * batch, _D), jnp.float32),
        mesh=plsc.VectorSubcoreMesh(core_axis_name="c", subcore_axis_name="s",
                                    num_cores=_NC),
        compiler_params=pltpu.CompilerParams(needs_layout_passes=False),
        scratch_types=[
            pltpu.VMEM((2 * (batch // _NW) // _D, _D), jnp.float32),
            pltpu.VMEM((8, _D), jnp.float32),
            pltpu.VMEM((2 * (batch // _NW) // _CROWS, _CROWS), jnp.int32),
            pltpu.VMEM((_NBUF, _CROWS, _D), jnp.float32),
            pltpu.VMEM_SHARED((2 * (_BINS + 4), _D), jnp.float32),
            [pltpu.SemaphoreType.DMA] * 3,
            [pltpu.SemaphoreType.DMA] * _NBUF,
            [pltpu.SemaphoreType.DMA] * _NBUF,
        ],
    )
    return grid_kernel(packed)


def kernel(crs, lat_table, lon_table):
    batch = crs.shape[0]
    assert batch % (_NW * _CROWS // 2) == 0 and (2 * batch) % _D == 0
    lat_edges = jnp.linspace(_LAT_MIN, _LAT_MAX, _BINS - 1)
    lon_edges = jnp.linspace(_LON_MIN, _LON_MAX, _BINS - 1)
    edge_rows = jnp.zeros((2, _D), jnp.float32) \
        .at[0, :_BINS - 1].set(lat_edges) \
        .at[1, :_BINS - 1].set(lon_edges)
    pad4 = jnp.zeros((4, _D), jnp.float32)
    pad6 = jnp.zeros((6, _D), jnp.float32)
    packed = jnp.concatenate(
        [crs.reshape(2 * batch // _D, _D), lat_table, pad4, lon_table, pad4,
         edge_rows, pad6], axis=0)
    out = _sc_run(packed, batch)
    return out.reshape(batch, 2 * _D)
